# baseline (speedup 1.0000x reference)
import functools

import jax
import jax.numpy as jnp
from jax import lax
from jax.experimental import pallas as pl
from jax.experimental.pallas import tpu as pltpu
from jax.experimental.pallas import tpu_sc as plsc

NC = 2
NS = 16
NW = NC * NS
L = 16


def _tc_transpose(e2t):
    K, V = e2t.shape
    C = 25600
    G = 40
    Vp = C * G
    TAIL = V - (G - 1) * C
    TFULL = (TAIL // 128) * 128

    def body(in_hbm, tail_ref, out_ref, scratch, sem):
        i = pl.program_id(0)

        @pl.when(i < G - 1)
        def _full():
            cp = pltpu.make_async_copy(
                in_hbm.at[:, pl.ds(i * C, C)], scratch, sem)
            cp.start()
            cp.wait()

        @pl.when(i == G - 1)
        def _tail():
            cp = pltpu.make_async_copy(
                in_hbm.at[:, pl.ds((G - 1) * C, TFULL)],
                scratch.at[:, :TFULL], sem)
            cp.start()
            cp.wait()
            scratch[:, TFULL:TAIL] = tail_ref[...]

        out_ref[...] = scratch[...].T.reshape(C // 8, 8 * K)

    out = pl.pallas_call(
        body,
        grid=(G,),
        in_specs=[pl.BlockSpec(memory_space=pl.ANY),
                  pl.BlockSpec((K, TAIL - TFULL), lambda i: (0, 0))],
        out_specs=pl.BlockSpec((C // 8, 8 * K), lambda i: (i, 0)),
        out_shape=jax.ShapeDtypeStruct((Vp * K // (8 * K), 8 * K), jnp.float32),
        scratch_shapes=[pltpu.VMEM((K, C), jnp.float32),
                        pltpu.SemaphoreType.DMA],
    )(e2t, lax.slice(e2t, (0, (G - 1) * C + TFULL), (K, V)))
    return out.reshape(Vp, K)


def _sc_pool(idx_w, emb_one_flat, emb_second_row, *, B, F, K, BPW):
    mesh = plsc.VectorSubcoreMesh(core_axis_name="c", subcore_axis_name="s")

    @functools.partial(
        pl.kernel,
        out_type=(
            jax.ShapeDtypeStruct((B, K), jnp.float32),
            jax.ShapeDtypeStruct((B, K), jnp.float32),
            jax.ShapeDtypeStruct((B,), jnp.float32),
        ),
        mesh=mesh,
        scratch_types=[
            pltpu.VMEM((F, BPW), jnp.int32),
            pltpu.VMEM((F, BPW, K), jnp.float32),
            pltpu.VMEM((F, BPW), jnp.float32),
            pltpu.VMEM((BPW, K), jnp.float32),
            pltpu.VMEM((BPW, K), jnp.float32),
            pltpu.VMEM((BPW,), jnp.float32),
            pltpu.SemaphoreType.DMA,
            pltpu.SemaphoreType.DMA,
        ],
        compiler_params=pltpu.CompilerParams(use_tc_tiling_on_sc=False),
    )
    def k(idx_hbm, emb1_hbm, emb2_hbm, sum_out, sq_out, one_out,
          idx_v, rows_v, one_v, sum_v, sq_v, ones_v, sem2, sem1):
        wid = lax.axis_index("s") * NC + lax.axis_index("c")
        base = wid * BPW
        pltpu.sync_copy(idx_hbm.at[wid], idx_v)
        cps = []
        for f in range(F):
            cps.append(pltpu.async_copy(emb2_hbm.at[idx_v.at[f]], rows_v.at[f], sem2))
            cps.append(pltpu.async_copy(emb1_hbm.at[idx_v.at[f]], one_v.at[f], sem1))
        for c in cps:
            c.wait()

        def body(j, carry):
            s = rows_v[0, j]
            q = s * s
            for f in range(1, F):
                v = rows_v[f, j]
                s = s + v
                q = q + v * v
            sum_v[j] = s
            sq_v[j] = q
            return carry

        lax.fori_loop(0, BPW, body, 0, unroll=False)

        for g in range(BPW // L):
            a = one_v[0, pl.ds(g * L, L)]
            for f in range(1, F):
                a = a + one_v[f, pl.ds(g * L, L)]
            ones_v[pl.ds(g * L, L)] = a

        pltpu.sync_copy(sum_v, sum_out.at[pl.ds(base, BPW)])
        pltpu.sync_copy(sq_v, sq_out.at[pl.ds(base, BPW)])
        pltpu.sync_copy(ones_v, one_out.at[pl.ds(base, BPW)])

    return k(idx_w, emb_one_flat, emb_second_row)


def _tc_combine(sum_vec, sq_vec, one_sum, dense_inputs, dense_one_row,
                dense_second_mat, zero_bias):
    B, K = sum_vec.shape

    def body(sum_ref, sq_ref, one_ref, dense_ref, d1_ref, ds2_ref, bias_ref,
             out_ref):
        dense = dense_ref[...]
        ds2 = ds2_ref[...]
        s = sum_ref[...] + jnp.dot(dense, ds2, preferred_element_type=jnp.float32)
        q = sq_ref[...] + jnp.dot(dense * dense, ds2 * ds2,
                                  preferred_element_type=jnp.float32)
        first = one_ref[...] + jnp.sum(dense * d1_ref[...], axis=1, keepdims=True)
        second = 0.5 * jnp.sum(s * s - q, axis=1, keepdims=True)
        out_ref[...] = jax.nn.sigmoid(first + second + bias_ref[0, 0])

    return pl.pallas_call(
        body,
        out_shape=jax.ShapeDtypeStruct((B, 1), jnp.float32),
    )(sum_vec, sq_vec, one_sum.reshape(B, 1), dense_inputs,
      dense_one_row, dense_second_mat, zero_bias.reshape(1, 1))


def kernel(sparse_inputs, dense_inputs, emb_one, emb_second, dense_one,
           dense_second, zero_bias):
    B, F = sparse_inputs.shape
    V, K = emb_second.shape
    BPW = B // NW
    idx_w = sparse_inputs.astype(jnp.int32).reshape(NW, BPW, F).transpose(0, 2, 1)
    sum_vec, sq_vec, one_sum = _sc_pool(
        idx_w, emb_one.reshape(V), emb_second, B=B, F=F, K=K, BPW=BPW)
    return _tc_combine(sum_vec, sq_vec, one_sum, dense_inputs,
                       dense_one.reshape(1, -1), dense_second.reshape(-1, K),
                       zero_bias)


# speedup vs baseline: 1.0016x; 1.0016x over previous
import functools

import jax
import jax.numpy as jnp
from jax import lax
from jax.experimental import pallas as pl
from jax.experimental.pallas import tpu as pltpu
from jax.experimental.pallas import tpu_sc as plsc

NC = 2
NS = 16
NW = NC * NS
L = 16


def _tc_transpose(e2t):
    K, V = e2t.shape
    C = 25600
    G = 40
    Vp = C * G
    TAIL = V - (G - 1) * C
    TFULL = (TAIL // 128) * 128

    def body(in_hbm, tail_ref, out_ref, scratch, sem):
        i = pl.program_id(0)

        @pl.when(i < G - 1)
        def _full():
            cp = pltpu.make_async_copy(
                in_hbm.at[:, pl.ds(i * C, C)], scratch, sem)
            cp.start()
            cp.wait()

        @pl.when(i == G - 1)
        def _tail():
            cp = pltpu.make_async_copy(
                in_hbm.at[:, pl.ds((G - 1) * C, TFULL)],
                scratch.at[:, :TFULL], sem)
            cp.start()
            cp.wait()
            scratch[:, TFULL:TAIL] = tail_ref[...]

        out_ref[...] = scratch[...].T.reshape(C // 8, 8 * K)

    out = pl.pallas_call(
        body,
        grid=(G,),
        in_specs=[pl.BlockSpec(memory_space=pl.ANY),
                  pl.BlockSpec((K, TAIL - TFULL), lambda i: (0, 0))],
        out_specs=pl.BlockSpec((C // 8, 8 * K), lambda i: (i, 0)),
        out_shape=jax.ShapeDtypeStruct((Vp * K // (8 * K), 8 * K), jnp.float32),
        scratch_shapes=[pltpu.VMEM((K, C), jnp.float32),
                        pltpu.SemaphoreType.DMA],
    )(e2t, lax.slice(e2t, (0, (G - 1) * C + TFULL), (K, V)))
    return out.reshape(Vp, K)


def _sc_pool(idx_w, emb_one_flat, emb_second_row, *, B, F, K, BPW):
    mesh = plsc.VectorSubcoreMesh(core_axis_name="c", subcore_axis_name="s")

    @functools.partial(
        pl.kernel,
        out_type=(
            jax.ShapeDtypeStruct((B, K), jnp.float32),
            jax.ShapeDtypeStruct((B, K), jnp.float32),
            jax.ShapeDtypeStruct((B,), jnp.float32),
        ),
        mesh=mesh,
        scratch_types=[
            pltpu.VMEM((F, BPW), jnp.int32),
            pltpu.VMEM((F, BPW, K), jnp.float32),
            pltpu.VMEM((F, BPW), jnp.float32),
            pltpu.VMEM((BPW, K), jnp.float32),
            pltpu.VMEM((BPW, K), jnp.float32),
            pltpu.VMEM((BPW,), jnp.float32),
            pltpu.SemaphoreType.DMA,
            pltpu.SemaphoreType.DMA,
        ],
        compiler_params=pltpu.CompilerParams(use_tc_tiling_on_sc=False),
    )
    def k(idx_hbm, emb1_hbm, emb2_hbm, sum_out, sq_out, one_out,
          idx_v, rows_v, one_v, sum_v, sq_v, ones_v, sem2, sem1):
        wid = lax.axis_index("s") * NC + lax.axis_index("c")
        base = wid * BPW
        pltpu.sync_copy(idx_hbm.at[wid], idx_v)
        cps = []
        for f in range(F):
            cps.append(pltpu.async_copy(emb2_hbm.at[idx_v.at[f]], rows_v.at[f], sem2))
            cps.append(pltpu.async_copy(emb1_hbm.at[idx_v.at[f]], one_v.at[f], sem1))
        for c in cps:
            c.wait()

        def body(j, carry):
            s = rows_v[0, j]
            q = s * s
            for f in range(1, F):
                v = rows_v[f, j]
                s = s + v
                q = q + v * v
            sum_v[j] = s
            sq_v[j] = q
            return carry

        lax.fori_loop(0, BPW, body, 0, unroll=False)

        for g in range(BPW // L):
            a = one_v[0, pl.ds(g * L, L)]
            for f in range(1, F):
                a = a + one_v[f, pl.ds(g * L, L)]
            ones_v[pl.ds(g * L, L)] = a

        pltpu.sync_copy(sum_v, sum_out.at[pl.ds(base, BPW)])
        pltpu.sync_copy(sq_v, sq_out.at[pl.ds(base, BPW)])
        pltpu.sync_copy(ones_v, one_out.at[pl.ds(base, BPW)])

    return k(idx_w, emb_one_flat, emb_second_row)


def _tc_combine(sum_vec, sq_vec, one_sum, dense_inputs, dense_one_row,
                dense_second_mat, zero_bias):
    B, K = sum_vec.shape

    def body(sum_ref, sq_ref, one_ref, dense_ref, d1_ref, ds2_ref, bias_ref,
             out_ref):
        dense = dense_ref[...]
        ds2 = ds2_ref[...]
        s = sum_ref[...] + jnp.dot(dense, ds2, preferred_element_type=jnp.float32)
        q = sq_ref[...] + jnp.dot(dense * dense, ds2 * ds2,
                                  preferred_element_type=jnp.float32)
        first = one_ref[...] + jnp.sum(dense * d1_ref[...], axis=1, keepdims=True)
        second = 0.5 * jnp.sum(s * s - q, axis=1, keepdims=True)
        out_ref[...] = jax.nn.sigmoid(first + second + bias_ref[0, 0])

    return pl.pallas_call(
        body,
        out_shape=jax.ShapeDtypeStruct((B, 1), jnp.float32),
    )(sum_vec, sq_vec, one_sum.reshape(B, 1), dense_inputs,
      dense_one_row, dense_second_mat, zero_bias.reshape(1, 1))


def kernel(sparse_inputs, dense_inputs, emb_one, emb_second, dense_one,
           dense_second, zero_bias):
    B, F = sparse_inputs.shape
    V, K = emb_second.shape
    BPW = B // NW
    idx_w = sparse_inputs.astype(jnp.int32).reshape(NW, BPW, F).transpose(0, 2, 1)
    e2lin = lax.optimization_barrier(
        emb_second.reshape(V * K // 128, 128)).reshape(V, K)
    sum_vec, sq_vec, one_sum = _sc_pool(
        idx_w, emb_one.reshape(V), e2lin, B=B, F=F, K=K, BPW=BPW)
    return _tc_combine(sum_vec, sq_vec, one_sum, dense_inputs,
                       dense_one.reshape(1, -1), dense_second.reshape(-1, K),
                       zero_bias)
